# baseline (speedup 1.0000x reference)
import functools

import jax
import jax.numpy as jnp
from jax import lax
from jax.experimental import pallas as pl
from jax.experimental.pallas import tpu as pltpu
from jax.experimental.pallas import tpu_sc as plsc

_NC = 2
_NS = 16
_NW = _NC * _NS

_BATCH = 16384
_HEADS = 26
_DIM = 32
_N = _BATCH * _HEADS
_PER_W = _N // _NW
_CH = 128
_NCHUNK = _PER_W // _CH
_LANES = 16

_VOCAB = 2600000
_TBLK = 2048
_TGRID = -(-_VOCAB // _TBLK)


def _transpose_body(in_ref, out_ref):
    x = in_ref[...]
    x = x.reshape(_DIM, _TBLK // 4, 4)
    x = jnp.transpose(x, (1, 2, 0))
    out_ref[...] = x.reshape(_TBLK // 4, 128)


def _retile_table(table_t):
    return pl.pallas_call(
        _transpose_body,
        grid=(_TGRID,),
        in_specs=[pl.BlockSpec((_DIM, _TBLK), lambda j: (0, j))],
        out_specs=pl.BlockSpec((_TBLK // 4, 128), lambda j: (j, 0)),
        out_shape=jax.ShapeDtypeStruct((_VOCAB * _DIM // 128, 128),
                                       jnp.float32),
    )(table_t)


def _body(hash_hbm, table_hbm, off_hbm, out_hbm,
          hash_v, idx_v, col_v, off_v, g_buf, o_buf,
          hsem, gsem0, gsem1, wsem0, wsem1):
    wid = lax.axis_index("s") * _NC + lax.axis_index("c")
    base = wid * _PER_W

    pltpu.sync_copy(off_hbm, off_v)

    def flat_pos(c):
        f = base + c * _CH
        b0 = pl.multiple_of(f & (_BATCH - 1), _CH)
        return f >> 14, b0

    def h_copy(c, s):
        h, b0 = flat_pos(c)
        return pltpu.make_async_copy(
            hash_hbm.at[h].at[pl.ds(b0, _CH)], hash_v.at[s], hsem)

    def g_copy(c, s):
        return pltpu.make_async_copy(
            table_hbm.at[idx_v.at[s]], g_buf.at[s], gsems[s])

    def w_copy(c, s):
        h, b0 = flat_pos(c)
        return pltpu.make_async_copy(
            o_buf.at[s], out_hbm.at[h].at[:, pl.ds(b0, _CH)], wsems[s])

    gsems = (gsem0, gsem1)
    wsems = (wsem0, wsem1)
    d_vec = lax.iota(jnp.int32, _LANES)

    def compute_ids(c, s):
        h, _ = flat_pos(c)
        off16 = off_v[h, pl.ds(0, _LANES)]

        def _add(i, carry):
            v = hash_v[s, pl.ds(i * _LANES, _LANES)] + off16
            col_v[s, pl.ds(i * _LANES, _LANES)] = (v & 3) * _DIM
            idx_v[s, pl.ds(i * _LANES, _LANES)] = lax.shift_right_logical(v, 2)
            return carry

        lax.fori_loop(0, _CH // _LANES, _add, 0)

    def rearrange(c, s):
        def _rb(rb, carry):
            colv = col_v[s, pl.ds(rb * _LANES, _LANES)]
            for t in range(_LANES):
                col = colv[t]
                rr = rb * _LANES + t
                src = g_buf.at[s].at[rr]
                v0 = src[pl.ds(col, _LANES)]
                v1 = src[pl.ds(col + _LANES, _LANES)]
                b_vec = jnp.zeros((_LANES,), jnp.int32) + rr
                plsc.store_scatter(o_buf.at[s], [d_vec, b_vec], v0)
                plsc.store_scatter(o_buf.at[s], [d_vec + _LANES, b_vec], v1)
            return carry

        lax.fori_loop(0, _CH // _LANES, _rb, 0)

    h_copy(0, 0).start()

    def _pair(c2, carry):
        c0 = c2 * 2
        c1 = c0 + 1

        h_copy(c0, 0).wait()
        h_copy(c1, 1).start()
        compute_ids(c0, 0)
        g_copy(c0, 0).start()

        h_copy(c1, 1).wait()

        @pl.when(c2 < _NCHUNK // 2 - 1)
        def _():
            h_copy(c0 + 2, 0).start()

        compute_ids(c1, 1)
        g_copy(c0, 0).wait()
        g_copy(c1, 1).start()

        @pl.when(c2 > 0)
        def _():
            w_copy(c0 - 2, 0).wait()

        rearrange(c0, 0)
        w_copy(c0, 0).start()

        g_copy(c1, 1).wait()

        @pl.when(c2 > 0)
        def _():
            w_copy(c1 - 2, 1).wait()

        rearrange(c1, 1)
        w_copy(c1, 1).start()
        return carry

    lax.fori_loop(0, _NCHUNK // 2, _pair, 0)
    w_copy(_NCHUNK - 2, 0).wait()
    w_copy(_NCHUNK - 1, 1).wait()


def _gather(hash_t, table128, off128):
    mesh = plsc.VectorSubcoreMesh(core_axis_name="c", subcore_axis_name="s")
    k = functools.partial(
        pl.kernel,
        mesh=mesh,
        out_type=jax.ShapeDtypeStruct((_HEADS, _DIM, _BATCH), jnp.float32),
        scratch_types=[
            pltpu.VMEM((2, _CH), jnp.int32),
            pltpu.VMEM((2, _CH), jnp.int32),
            pltpu.VMEM((2, _CH), jnp.int32),
            pltpu.VMEM((_HEADS, 128), jnp.int32),
            pltpu.VMEM((2, _CH, 128), jnp.float32),
            pltpu.VMEM((2, _DIM, _CH), jnp.float32),
            pltpu.SemaphoreType.DMA,
            pltpu.SemaphoreType.DMA,
            pltpu.SemaphoreType.DMA,
            pltpu.SemaphoreType.DMA,
            pltpu.SemaphoreType.DMA,
        ],
        compiler_params=pltpu.CompilerParams(needs_layout_passes=False),
    )(_body)
    return k(hash_t, table128, off128)


def kernel(hash_ids, table, offsets):
    hash_t = hash_ids.T
    table_t = table.T
    table128 = _retile_table(table_t)
    off128 = jnp.tile(offsets[:, None], (1, 128))
    out = _gather(hash_t, table128, off128)
    return out.transpose(2, 0, 1)


# speedup vs baseline: 1.0033x; 1.0033x over previous
import functools

import jax
import jax.numpy as jnp
from jax import lax
from jax.experimental import pallas as pl
from jax.experimental.pallas import tpu as pltpu
from jax.experimental.pallas import tpu_sc as plsc

_NC = 2
_NS = 16
_NW = _NC * _NS

_BATCH = 16384
_HEADS = 26
_DIM = 32
_N = _BATCH * _HEADS
_PER_W = _N // _NW
_CH = 128
_NCHUNK = _PER_W // _CH
_LANES = 16

_VOCAB = 2600000
_TBLK = 2048
_TGRID = -(-_VOCAB // _TBLK)


def _transpose_body(in_ref, out_ref):
    x = in_ref[...]
    x = x.reshape(_DIM, _TBLK // 4, 4)
    x = jnp.transpose(x, (1, 2, 0))
    out_ref[...] = x.reshape(_TBLK // 4, 128)


def _retile_table(table_t):
    return pl.pallas_call(
        _transpose_body,
        grid=(_TGRID,),
        in_specs=[pl.BlockSpec((_DIM, _TBLK), lambda j: (0, j))],
        out_specs=pl.BlockSpec((_TBLK // 4, 128), lambda j: (j, 0)),
        out_shape=jax.ShapeDtypeStruct((_VOCAB * _DIM // 128, 128),
                                       jnp.float32),
    )(table_t)


def _body(hash_hbm, table_hbm, off_hbm, out_hbm,
          hash_v, idx_v, col_v, off_v, g_buf, o_buf,
          hsem, gsem0, gsem1, wsem0, wsem1):
    wid = lax.axis_index("s") * _NC + lax.axis_index("c")
    base = wid * _PER_W

    pltpu.sync_copy(off_hbm, off_v)

    def flat_pos(c):
        f = base + c * _CH
        b0 = pl.multiple_of(f & (_BATCH - 1), _CH)
        return f >> 14, b0

    def h_copy(c, s):
        h, b0 = flat_pos(c)
        return pltpu.make_async_copy(
            hash_hbm.at[h].at[pl.ds(b0, _CH)], hash_v.at[s], hsem)

    def g_copy(c, s):
        return pltpu.make_async_copy(
            table_hbm.at[idx_v.at[s]], g_buf.at[s], gsems[s])

    def w_copy(c, s):
        h, b0 = flat_pos(c)
        return pltpu.make_async_copy(
            o_buf.at[s], out_hbm.at[h].at[:, pl.ds(b0, _CH)], wsems[s])

    gsems = (gsem0, gsem1)
    wsems = (wsem0, wsem1)
    d_vec = lax.iota(jnp.int32, _LANES)

    def compute_ids(c, s):
        h, _ = flat_pos(c)
        off16 = off_v[h, pl.ds(0, _LANES)]

        def _add(i, carry):
            v = hash_v[s, pl.ds(i * _LANES, _LANES)] + off16
            col_v[s, pl.ds(i * _LANES, _LANES)] = (v & 3) * _DIM
            idx_v[s, pl.ds(i * _LANES, _LANES)] = lax.shift_right_logical(v, 2)
            return carry

        lax.fori_loop(0, _CH // _LANES, _add, 0)

    def rearrange(c, s):
        def _rb(rb, carry):
            colv = col_v[s, pl.ds(rb * _LANES, _LANES)]
            for t in range(_LANES):
                col = colv[t]
                rr = rb * _LANES + t
                src = g_buf.at[s].at[rr]
                v0 = src[pl.ds(col, _LANES)]
                v1 = src[pl.ds(col + _LANES, _LANES)]
                b_vec = jnp.zeros((_LANES,), jnp.int32) + rr
                plsc.store_scatter(o_buf.at[s], [d_vec, b_vec], v0)
                plsc.store_scatter(o_buf.at[s], [d_vec + _LANES, b_vec], v1)
            return carry

        lax.fori_loop(0, _CH // _LANES, _rb, 0)

    h_copy(0, 0).start()

    def _pair(c2, carry):
        c0 = c2 * 2
        c1 = c0 + 1

        h_copy(c0, 0).wait()
        h_copy(c1, 1).start()
        compute_ids(c0, 0)
        g_copy(c0, 0).start()

        h_copy(c1, 1).wait()

        @pl.when(c2 < _NCHUNK // 2 - 1)
        def _():
            h_copy(c0 + 2, 0).start()

        compute_ids(c1, 1)
        g_copy(c0, 0).wait()
        g_copy(c1, 1).start()

        @pl.when(c2 > 0)
        def _():
            w_copy(c0 - 2, 0).wait()

        rearrange(c0, 0)
        w_copy(c0, 0).start()

        g_copy(c1, 1).wait()

        @pl.when(c2 > 0)
        def _():
            w_copy(c1 - 2, 1).wait()

        rearrange(c1, 1)
        w_copy(c1, 1).start()
        return carry

    lax.fori_loop(0, _NCHUNK // 2, _pair, 0)
    w_copy(_NCHUNK - 2, 0).wait()
    w_copy(_NCHUNK - 1, 1).wait()


def _gather(hash_t, table128, off128):
    mesh = plsc.VectorSubcoreMesh(core_axis_name="c", subcore_axis_name="s",
                                  num_cores=_NC)
    k = functools.partial(
        pl.kernel,
        mesh=mesh,
        out_type=jax.ShapeDtypeStruct((_HEADS, _DIM, _BATCH), jnp.float32),
        scratch_types=[
            pltpu.VMEM((2, _CH), jnp.int32),
            pltpu.VMEM((2, _CH), jnp.int32),
            pltpu.VMEM((2, _CH), jnp.int32),
            pltpu.VMEM((_HEADS, 128), jnp.int32),
            pltpu.VMEM((2, _CH, 128), jnp.float32),
            pltpu.VMEM((2, _DIM, _CH), jnp.float32),
            pltpu.SemaphoreType.DMA,
            pltpu.SemaphoreType.DMA,
            pltpu.SemaphoreType.DMA,
            pltpu.SemaphoreType.DMA,
            pltpu.SemaphoreType.DMA,
        ],
        compiler_params=pltpu.CompilerParams(needs_layout_passes=False),
    )(_body)
    return k(hash_t, table128, off128)


def kernel(hash_ids, table, offsets):
    hash_t = hash_ids.T
    table_t = table.T
    table128 = _retile_table(table_t)
    off128 = jnp.tile(offsets[:, None], (1, 128))
    out = _gather(hash_t, table128, off128)
    return out.transpose(2, 0, 1)


# speedup vs baseline: 4.7530x; 4.7375x over previous
import functools

import jax
import jax.numpy as jnp
from jax import lax
from jax.experimental import pallas as pl
from jax.experimental.pallas import tpu as pltpu
from jax.experimental.pallas import tpu_sc as plsc

_NC = 2
_NS = 16
_NW = _NC * _NS

_BATCH = 16384
_HEADS = 26
_DIM = 32
_N = _BATCH * _HEADS
_PER_W = _N // _NW
_CH = 128
_NCHUNK = _PER_W // _CH
_LANES = 16

_VOCAB = 2600000
_TBLK = 2048
_TGRID = -(-_VOCAB // _TBLK)


def _transpose_body(in_ref, out_ref):
    x = in_ref[...]
    q = _TBLK // 4
    parts = [x[:, i * q:(i + 1) * q].T for i in range(4)]
    out_ref[...] = jnp.concatenate(parts, axis=1)


def _retile_table(table_t):
    return pl.pallas_call(
        _transpose_body,
        grid=(_TGRID,),
        in_specs=[pl.BlockSpec((_DIM, _TBLK), lambda j: (0, j))],
        out_specs=pl.BlockSpec((_TBLK // 4, 128), lambda j: (j, 0)),
        out_shape=jax.ShapeDtypeStruct((_TGRID * _TBLK // 4, 128),
                                       jnp.float32),
    )(table_t)


def _body(hash_hbm, table_hbm, off_hbm, out_hbm,
          hash_v, idx_v, col_v, off_v, g_buf, o_buf,
          hsem, gsem0, gsem1, wsem0, wsem1):
    wid = lax.axis_index("s") * _NC + lax.axis_index("c")
    base = wid * _PER_W

    pltpu.sync_copy(off_hbm, off_v)

    def flat_pos(c):
        f = base + c * _CH
        b0 = pl.multiple_of(f & (_BATCH - 1), _CH)
        return f >> 14, b0

    def h_copy(c, s):
        h, b0 = flat_pos(c)
        return pltpu.make_async_copy(
            hash_hbm.at[h].at[pl.ds(b0, _CH)], hash_v.at[s], hsem)

    def g_copy(c, s):
        return pltpu.make_async_copy(
            table_hbm.at[idx_v.at[s]], g_buf.at[s], gsems[s])

    def w_copy(c, s):
        h, b0 = flat_pos(c)
        return pltpu.make_async_copy(
            o_buf.at[s], out_hbm.at[h].at[:, pl.ds(b0, _CH)], wsems[s])

    gsems = (gsem0, gsem1)
    wsems = (wsem0, wsem1)
    d_vec = lax.iota(jnp.int32, _LANES)

    def compute_ids(c, s):
        h, _ = flat_pos(c)
        off16 = off_v[h, pl.ds(0, _LANES)]

        def _add(i, carry):
            v = hash_v[s, pl.ds(i * _LANES, _LANES)] + off16
            col_v[s, pl.ds(i * _LANES, _LANES)] = (
                lax.shift_right_logical(v, 9) & 3) * _DIM
            idx_v[s, pl.ds(i * _LANES, _LANES)] = (
                (lax.shift_right_logical(v, 11) * 512) | (v & 511))
            return carry

        lax.fori_loop(0, _CH // _LANES, _add, 0)

    def rearrange(c, s):
        def _rb(rb, carry):
            colv = col_v[s, pl.ds(rb * _LANES, _LANES)]
            for t in range(_LANES):
                col = colv[t]
                rr = rb * _LANES + t
                src = g_buf.at[s].at[rr]
                v0 = src[pl.ds(col, _LANES)]
                v1 = src[pl.ds(col + _LANES, _LANES)]
                b_vec = jnp.zeros((_LANES,), jnp.int32) + rr
                plsc.store_scatter(o_buf.at[s], [d_vec, b_vec], v0)
                plsc.store_scatter(o_buf.at[s], [d_vec + _LANES, b_vec], v1)
            return carry

        lax.fori_loop(0, _CH // _LANES, _rb, 0)

    h_copy(0, 0).start()

    def _pair(c2, carry):
        c0 = c2 * 2
        c1 = c0 + 1

        h_copy(c0, 0).wait()
        h_copy(c1, 1).start()
        compute_ids(c0, 0)
        g_copy(c0, 0).start()

        h_copy(c1, 1).wait()

        @pl.when(c2 < _NCHUNK // 2 - 1)
        def _():
            h_copy(c0 + 2, 0).start()

        compute_ids(c1, 1)
        g_copy(c0, 0).wait()
        g_copy(c1, 1).start()

        @pl.when(c2 > 0)
        def _():
            w_copy(c0 - 2, 0).wait()

        rearrange(c0, 0)
        w_copy(c0, 0).start()

        g_copy(c1, 1).wait()

        @pl.when(c2 > 0)
        def _():
            w_copy(c1 - 2, 1).wait()

        rearrange(c1, 1)
        w_copy(c1, 1).start()
        return carry

    lax.fori_loop(0, _NCHUNK // 2, _pair, 0)
    w_copy(_NCHUNK - 2, 0).wait()
    w_copy(_NCHUNK - 1, 1).wait()


def _gather(hash_t, table128, off128):
    mesh = plsc.VectorSubcoreMesh(core_axis_name="c", subcore_axis_name="s",
                                  num_cores=_NC)
    k = functools.partial(
        pl.kernel,
        mesh=mesh,
        out_type=jax.ShapeDtypeStruct((_HEADS, _DIM, _BATCH), jnp.float32),
        scratch_types=[
            pltpu.VMEM((2, _CH), jnp.int32),
            pltpu.VMEM((2, _CH), jnp.int32),
            pltpu.VMEM((2, _CH), jnp.int32),
            pltpu.VMEM((_HEADS, 128), jnp.int32),
            pltpu.VMEM((2, _CH, 128), jnp.float32),
            pltpu.VMEM((2, _DIM, _CH), jnp.float32),
            pltpu.SemaphoreType.DMA,
            pltpu.SemaphoreType.DMA,
            pltpu.SemaphoreType.DMA,
            pltpu.SemaphoreType.DMA,
            pltpu.SemaphoreType.DMA,
        ],
        compiler_params=pltpu.CompilerParams(needs_layout_passes=False),
    )(_body)
    return k(hash_t, table128, off128)


def kernel(hash_ids, table, offsets):
    hash_t = hash_ids.T
    table_t = table.T
    table128 = _retile_table(table_t)
    off128 = jnp.tile(offsets[:, None], (1, 128))
    out = _gather(hash_t, table128, off128)
    return out.transpose(2, 0, 1)


# speedup vs baseline: 4.9077x; 1.0325x over previous
import functools

import jax
import jax.numpy as jnp
from jax import lax
from jax.experimental import pallas as pl
from jax.experimental.pallas import tpu as pltpu
from jax.experimental.pallas import tpu_sc as plsc

_NC = 2
_NS = 16
_NW = _NC * _NS

_BATCH = 16384
_HEADS = 26
_DIM = 32
_N = _BATCH * _HEADS
_PER_W = _N // _NW
_CH = 128
_NCHUNK = _PER_W // _CH
_LANES = 16

_VOCAB = 2600000
_TBLK = 2048
_TGRID = -(-_VOCAB // _TBLK)


def _transpose_body(in_ref, e_ref, out_ref):
    x = in_ref[...]
    e = e_ref[...]
    q = _TBLK // 4
    acc = jnp.zeros((q, 128), jnp.float32)
    for i in range(4):
        acc = acc + lax.dot_general(
            x[:, i * q:(i + 1) * q], e[:, i * 128:(i + 1) * 128],
            (((0,), (0,)), ((), ())), preferred_element_type=jnp.float32)
    out_ref[...] = acc


def _retile_table(table_t):
    eye = jnp.eye(_DIM, dtype=jnp.float32)
    e = jnp.zeros((_DIM, 4, 128), jnp.float32)
    for i in range(4):
        e = e.at[:, i, i * _DIM:(i + 1) * _DIM].set(eye)
    e = e.reshape(_DIM, 512)
    return pl.pallas_call(
        _transpose_body,
        grid=(_TGRID,),
        in_specs=[pl.BlockSpec((_DIM, _TBLK), lambda j: (0, j)),
                  pl.BlockSpec((_DIM, 512), lambda j: (0, 0))],
        out_specs=pl.BlockSpec((_TBLK // 4, 128), lambda j: (j, 0)),
        out_shape=jax.ShapeDtypeStruct((_TGRID * _TBLK // 4, 128),
                                       jnp.float32),
    )(table_t, e)


def _body(hash_hbm, table_hbm, off_hbm, out_hbm,
          hash_v, idx_v, col_v, off_v, g_buf, o_buf,
          hsem, gsem0, gsem1, wsem0, wsem1):
    wid = lax.axis_index("s") * _NC + lax.axis_index("c")
    base = wid * _PER_W

    pltpu.sync_copy(off_hbm, off_v)

    def flat_pos(c):
        f = base + c * _CH
        b0 = pl.multiple_of(f & (_BATCH - 1), _CH)
        return f >> 14, b0

    def h_copy(c, s):
        h, b0 = flat_pos(c)
        return pltpu.make_async_copy(
            hash_hbm.at[h].at[pl.ds(b0, _CH)], hash_v.at[s], hsem)

    def g_copy(c, s):
        return pltpu.make_async_copy(
            table_hbm.at[idx_v.at[s]], g_buf.at[s], gsems[s])

    def w_copy(c, s):
        h, b0 = flat_pos(c)
        return pltpu.make_async_copy(
            o_buf.at[s], out_hbm.at[h].at[:, pl.ds(b0, _CH)], wsems[s])

    gsems = (gsem0, gsem1)
    wsems = (wsem0, wsem1)
    d_vec = lax.iota(jnp.int32, _LANES)

    def compute_ids(c, s):
        h, _ = flat_pos(c)
        off16 = off_v[h, pl.ds(0, _LANES)]

        def _add(i, carry):
            v = hash_v[s, pl.ds(i * _LANES, _LANES)] + off16
            col_v[s, pl.ds(i * _LANES, _LANES)] = (
                lax.shift_right_logical(v, 9) & 3) * _DIM
            idx_v[s, pl.ds(i * _LANES, _LANES)] = (
                (lax.shift_right_logical(v, 11) * 512) | (v & 511))
            return carry

        lax.fori_loop(0, _CH // _LANES, _add, 0)

    def rearrange(c, s):
        def _rb(rb, carry):
            colv = col_v[s, pl.ds(rb * _LANES, _LANES)]
            for t in range(_LANES):
                col = colv[t]
                rr = rb * _LANES + t
                src = g_buf.at[s].at[rr]
                v0 = src[pl.ds(col, _LANES)]
                v1 = src[pl.ds(col + _LANES, _LANES)]
                b_vec = jnp.zeros((_LANES,), jnp.int32) + rr
                plsc.store_scatter(o_buf.at[s], [d_vec, b_vec], v0)
                plsc.store_scatter(o_buf.at[s], [d_vec + _LANES, b_vec], v1)
            return carry

        lax.fori_loop(0, _CH // _LANES, _rb, 0)

    h_copy(0, 0).start()

    def _pair(c2, carry):
        c0 = c2 * 2
        c1 = c0 + 1

        h_copy(c0, 0).wait()
        h_copy(c1, 1).start()
        compute_ids(c0, 0)
        g_copy(c0, 0).start()

        h_copy(c1, 1).wait()

        @pl.when(c2 < _NCHUNK // 2 - 1)
        def _():
            h_copy(c0 + 2, 0).start()

        compute_ids(c1, 1)
        g_copy(c0, 0).wait()
        g_copy(c1, 1).start()

        @pl.when(c2 > 0)
        def _():
            w_copy(c0 - 2, 0).wait()

        rearrange(c0, 0)
        w_copy(c0, 0).start()

        g_copy(c1, 1).wait()

        @pl.when(c2 > 0)
        def _():
            w_copy(c1 - 2, 1).wait()

        rearrange(c1, 1)
        w_copy(c1, 1).start()
        return carry

    lax.fori_loop(0, _NCHUNK // 2, _pair, 0)
    w_copy(_NCHUNK - 2, 0).wait()
    w_copy(_NCHUNK - 1, 1).wait()


def _gather(hash_t, table128, off128):
    mesh = plsc.VectorSubcoreMesh(core_axis_name="c", subcore_axis_name="s",
                                  num_cores=_NC)
    k = functools.partial(
        pl.kernel,
        mesh=mesh,
        out_type=jax.ShapeDtypeStruct((_HEADS, _DIM, _BATCH), jnp.float32),
        scratch_types=[
            pltpu.VMEM((2, _CH), jnp.int32),
            pltpu.VMEM((2, _CH), jnp.int32),
            pltpu.VMEM((2, _CH), jnp.int32),
            pltpu.VMEM((_HEADS, 128), jnp.int32),
            pltpu.VMEM((2, _CH, 128), jnp.float32),
            pltpu.VMEM((2, _DIM, _CH), jnp.float32),
            pltpu.SemaphoreType.DMA,
            pltpu.SemaphoreType.DMA,
            pltpu.SemaphoreType.DMA,
            pltpu.SemaphoreType.DMA,
            pltpu.SemaphoreType.DMA,
        ],
        compiler_params=pltpu.CompilerParams(needs_layout_passes=False),
    )(_body)
    return k(hash_t, table128, off128)


def kernel(hash_ids, table, offsets):
    hash_t = hash_ids.T
    table_t = table.T
    table128 = _retile_table(table_t)
    off128 = jnp.tile(offsets[:, None], (1, 128))
    out = _gather(hash_t, table128, off128)
    return out.transpose(2, 0, 1)


# speedup vs baseline: 7.8898x; 1.6076x over previous
import functools

import jax
import jax.numpy as jnp
from jax import lax
from jax.experimental import pallas as pl
from jax.experimental.pallas import tpu as pltpu
from jax.experimental.pallas import tpu_sc as plsc

_NC = 2
_NS = 16
_NW = _NC * _NS

_BATCH = 16384
_HEADS = 26
_DIM = 32
_N = _BATCH * _HEADS
_PER_W = _N // _NW
_CH = 128
_NCHUNK = _PER_W // _CH
_LANES = 16

_VOCAB = 2600000
_TBLK = 8192
_TGRID = -(-_VOCAB // _TBLK)


def _transpose_body(in_ref, e_ref, out_ref):
    x = in_ref[...]
    e = e_ref[...]
    q = _TBLK // 4
    acc = jnp.zeros((q, 128), jnp.float32)
    for i in range(4):
        acc = acc + lax.dot_general(
            x[:, i * q:(i + 1) * q], e[:, i * 128:(i + 1) * 128],
            (((0,), (0,)), ((), ())), preferred_element_type=jnp.float32)
    out_ref[...] = acc


def _retile_table(table_t):
    eye = jnp.eye(_DIM, dtype=jnp.float32)
    e = jnp.zeros((_DIM, 4, 128), jnp.float32)
    for i in range(4):
        e = e.at[:, i, i * _DIM:(i + 1) * _DIM].set(eye)
    e = e.reshape(_DIM, 512)
    return pl.pallas_call(
        _transpose_body,
        grid=(_TGRID,),
        in_specs=[pl.BlockSpec((_DIM, _TBLK), lambda j: (0, j)),
                  pl.BlockSpec((_DIM, 512), lambda j: (0, 0))],
        out_specs=pl.BlockSpec((_TBLK // 4, 128), lambda j: (j, 0)),
        out_shape=jax.ShapeDtypeStruct((_TGRID * _TBLK // 4, 128),
                                       jnp.float32),
    )(table_t, e)


def _body(hash_hbm, table_hbm, off_hbm, out_hbm,
          hash_v, idx_v, col_v, off_v, g_buf, o_buf,
          hsem, gsem0, gsem1, wsem0, wsem1):
    wid = lax.axis_index("s") * _NC + lax.axis_index("c")
    base = wid * _PER_W

    pltpu.sync_copy(off_hbm, off_v)

    def flat_pos(c):
        f = base + c * _CH
        b0 = pl.multiple_of(f & (_BATCH - 1), _CH)
        return f >> 14, b0

    def h_copy(c, s):
        h, b0 = flat_pos(c)
        return pltpu.make_async_copy(
            hash_hbm.at[h].at[pl.ds(b0, _CH)], hash_v.at[s], hsem)

    def g_copy(c, s):
        return pltpu.make_async_copy(
            table_hbm.at[idx_v.at[s]], g_buf.at[s], gsems[s])

    def w_copy(c, s):
        h, b0 = flat_pos(c)
        return pltpu.make_async_copy(
            o_buf.at[s], out_hbm.at[h].at[:, pl.ds(b0, _CH)], wsems[s])

    gsems = (gsem0, gsem1)
    wsems = (wsem0, wsem1)
    d_vec = lax.iota(jnp.int32, _LANES)

    def compute_ids(c, s):
        h, _ = flat_pos(c)
        off16 = off_v[h, pl.ds(0, _LANES)]

        def _add(i, carry):
            v = hash_v[s, pl.ds(i * _LANES, _LANES)] + off16
            col_v[s, pl.ds(i * _LANES, _LANES)] = (
                lax.shift_right_logical(v, 11) & 3) * _DIM
            idx_v[s, pl.ds(i * _LANES, _LANES)] = (
                (lax.shift_right_logical(v, 13) * 2048) | (v & 2047))
            return carry

        lax.fori_loop(0, _CH // _LANES, _add, 0)

    def rearrange(c, s):
        def _rb(rb, carry):
            colv = col_v[s, pl.ds(rb * _LANES, _LANES)]
            for t in range(_LANES):
                col = colv[t]
                rr = rb * _LANES + t
                src = g_buf.at[s].at[rr]
                v0 = src[pl.ds(col, _LANES)]
                v1 = src[pl.ds(col + _LANES, _LANES)]
                b_vec = jnp.zeros((_LANES,), jnp.int32) + rr
                plsc.store_scatter(o_buf.at[s], [d_vec, b_vec], v0)
                plsc.store_scatter(o_buf.at[s], [d_vec + _LANES, b_vec], v1)
            return carry

        lax.fori_loop(0, _CH // _LANES, _rb, 0)

    h_copy(0, 0).start()

    def _pair(c2, carry):
        c0 = c2 * 2
        c1 = c0 + 1

        h_copy(c0, 0).wait()
        h_copy(c1, 1).start()
        compute_ids(c0, 0)
        g_copy(c0, 0).start()

        h_copy(c1, 1).wait()

        @pl.when(c2 < _NCHUNK // 2 - 1)
        def _():
            h_copy(c0 + 2, 0).start()

        compute_ids(c1, 1)
        g_copy(c0, 0).wait()
        g_copy(c1, 1).start()

        @pl.when(c2 > 0)
        def _():
            w_copy(c0 - 2, 0).wait()

        rearrange(c0, 0)
        w_copy(c0, 0).start()

        g_copy(c1, 1).wait()

        @pl.when(c2 > 0)
        def _():
            w_copy(c1 - 2, 1).wait()

        rearrange(c1, 1)
        w_copy(c1, 1).start()
        return carry

    lax.fori_loop(0, _NCHUNK // 2, _pair, 0)
    w_copy(_NCHUNK - 2, 0).wait()
    w_copy(_NCHUNK - 1, 1).wait()


def _gather(hash_t, table128, off128):
    mesh = plsc.VectorSubcoreMesh(core_axis_name="c", subcore_axis_name="s",
                                  num_cores=_NC)
    k = functools.partial(
        pl.kernel,
        mesh=mesh,
        out_type=jax.ShapeDtypeStruct((_HEADS, _DIM, _BATCH), jnp.float32),
        scratch_types=[
            pltpu.VMEM((2, _CH), jnp.int32),
            pltpu.VMEM((2, _CH), jnp.int32),
            pltpu.VMEM((2, _CH), jnp.int32),
            pltpu.VMEM((_HEADS, 128), jnp.int32),
            pltpu.VMEM((2, _CH, 128), jnp.float32),
            pltpu.VMEM((2, _DIM, _CH), jnp.float32),
            pltpu.SemaphoreType.DMA,
            pltpu.SemaphoreType.DMA,
            pltpu.SemaphoreType.DMA,
            pltpu.SemaphoreType.DMA,
            pltpu.SemaphoreType.DMA,
        ],
        compiler_params=pltpu.CompilerParams(needs_layout_passes=False),
    )(_body)
    return k(hash_t, table128, off128)


def kernel(hash_ids, table, offsets):
    hash_t = hash_ids.T
    table_t = table.T
    table128 = _retile_table(table_t)
    off128 = jnp.tile(offsets[:, None], (1, 128))
    out = _gather(hash_t, table128, off128)
    return out.transpose(2, 0, 1)
